# Initial kernel scaffold; baseline (speedup 1.0000x reference)
import jax
import jax.numpy as jnp
from jax.experimental import pallas as pl


def kernel(predictions, targets):
    raise NotImplementedError("write your pallas kernel here")


# speedup vs baseline: 71.2390x; 71.2390x over previous
import jax
import jax.numpy as jnp
from jax.experimental import pallas as pl

_GO = -4.0
_GE = -1.0
_EGO = 0.01831563888873418
_EGE = 0.36787944117144233
_NEG = -1e30
_B = 16
_L = 256
_NSTEPS = 509


def _shift1(x):
    return jnp.concatenate(
        [jnp.full((x.shape[0], 1), _NEG, x.dtype), x[:, :-1]], axis=1)


def _rot1(x):
    return jnp.concatenate([x[:, -1:], x[:, :-1]], axis=1)


def _sel4(t, v):
    return jnp.where(t == 0, v[0],
           jnp.where(t == 1, v[1],
           jnp.where(t == 2, v[2], v[3])))


def _sw_kernel(predT_ref, v0_ref, out_ref):
    predT = predT_ref[...]
    v = v0_ref[...]
    lane = jax.lax.broadcasted_iota(jnp.int32, (_B, _L), 1)
    vmask = lane < (_L - 1)
    neg = jnp.full((_B, _L), _NEG, jnp.float32)

    p0 = [jnp.where(vmask, predT[p], neg) for p in range(4)]
    p1 = [jnp.where(vmask,
                    jnp.concatenate([predT[p][:, 1:], predT[p][:, :1]], axis=1),
                    neg) for p in range(4)]

    def body(d, carry):
        hp0, hp1, gp, gpp, qp, run_m, run_s, vc = carry
        vb = _rot1(vc)
        ld = lane - d
        mask = (ld <= 0) & (ld >= -254)
        smx = jnp.where(mask, _sel4(vc, p0), neg)
        xe = jnp.where(mask, _sel4(vb, p1), neg)

        gs = _shift1(gpp)
        mxa = jnp.maximum(gs, 0.0)
        a = smx + mxa + jnp.log(jnp.exp(gs - mxa) + jnp.exp(-mxa))
        t0 = hp0 + _GO
        t1 = hp1 + _GE
        mxr = jnp.maximum(t0, t1)
        r = mxr + jnp.log(jnp.exp(t0 - mxr) + jnp.exp(t1 - mxr))
        dn = _shift1(qp)

        mx3 = jnp.maximum(a, jnp.maximum(r, dn))
        ea = jnp.exp(a - mx3)
        er = jnp.exp(r - mx3)
        ed = jnp.exp(dn - mx3)
        g = mx3 + jnp.log(ea + er + ed)
        q = mx3 + jnp.log(_EGO * (ea + er) + _EGE * ed)

        z = g + xe
        mn = jnp.maximum(run_m, z)
        sn = run_s * jnp.exp(run_m - mn) + jnp.exp(z - mn)
        return (a, r, g, gp, q, mn, sn, vb)

    zi = predT[0] * 0.0
    negi = zi + _NEG
    init = (negi, negi, negi, negi, negi, negi, zi, v)
    out = jax.lax.fori_loop(0, _NSTEPS, body, init)
    run_m, run_s = out[5], out[6]

    mb = jnp.max(run_m, axis=1, keepdims=True)
    sb = jnp.sum(run_s * jnp.exp(run_m - mb), axis=1, keepdims=True)
    fin = mb + jnp.log(sb)
    out_ref[...] = jnp.full((1, 1), -jnp.sum(fin) * (1.0 / _B), jnp.float32)


def _prep(predictions, targets):
    predT = jnp.transpose(predictions.astype(jnp.float32), (2, 0, 1))
    t = targets.astype(jnp.int32)
    v0 = jnp.concatenate([t[:, :1], jnp.flip(t[:, 1:], axis=1)], axis=1)
    return predT, v0


@jax.jit
def kernel(predictions, targets):
    predT, v0 = _prep(predictions, targets)
    out = pl.pallas_call(
        _sw_kernel,
        out_shape=jax.ShapeDtypeStruct((1, 1), jnp.float32),
    )(predT, v0)
    return out[0, 0]


# speedup vs baseline: 86.3844x; 1.2126x over previous
import jax
import jax.numpy as jnp
from jax.experimental import pallas as pl

_GO = -4.0
_GE = -1.0
_EGO = 0.01831563888873418
_EGE = 0.36787944117144233
_NEG = -1e30
_B = 16
_L = 256
_NSTEPS = 509


def _shift1(x):
    return jnp.concatenate(
        [jnp.full((x.shape[0], 1), _NEG, x.dtype), x[:, :-1]], axis=1)


def _rot1(x):
    return jnp.concatenate([x[:, -1:], x[:, :-1]], axis=1)


def _sel4(t, v):
    return jnp.where(t == 0, v[0],
           jnp.where(t == 1, v[1],
           jnp.where(t == 2, v[2], v[3])))


def _sw_kernel(predT_ref, v0_ref, out_ref):
    predT = predT_ref[...]
    v = v0_ref[...]
    lane = jax.lax.broadcasted_iota(jnp.int32, (_B, _L), 1)
    vmask = lane < (_L - 1)
    neg = jnp.full((_B, _L), _NEG, jnp.float32)

    p0 = [jnp.where(vmask, predT[p], neg) for p in range(4)]
    p1 = [jnp.where(vmask,
                    jnp.concatenate([predT[p][:, 1:], predT[p][:, :1]], axis=1),
                    neg) for p in range(4)]

    def body(d, carry):
        hp0, hp1, gp, gpp, qp, run_m, run_s, vc = carry
        vb = _rot1(vc)
        ld = lane - d
        mask = (ld <= 0) & (ld >= -254)
        smx = jnp.where(mask, _sel4(vc, p0), neg)
        xe = jnp.where(mask, _sel4(vb, p1), neg)

        gs = _shift1(gpp)
        mxa = jnp.maximum(gs, 0.0)
        a = smx + mxa + jnp.log(jnp.exp(gs - mxa) + jnp.exp(-mxa))
        t0 = hp0 + _GO
        t1 = hp1 + _GE
        mxr = jnp.maximum(t0, t1)
        r = mxr + jnp.log(jnp.exp(t0 - mxr) + jnp.exp(t1 - mxr))
        dn = _shift1(qp)

        mx3 = jnp.maximum(a, jnp.maximum(r, dn))
        ea = jnp.exp(a - mx3)
        er = jnp.exp(r - mx3)
        ed = jnp.exp(dn - mx3)
        g = mx3 + jnp.log(ea + er + ed)
        q = mx3 + jnp.log(_EGO * (ea + er) + _EGE * ed)

        z = g + xe
        mn = jnp.maximum(run_m, z)
        sn = run_s * jnp.exp(run_m - mn) + jnp.exp(z - mn)
        return (a, r, g, gp, q, mn, sn, vb)

    zi = predT[0] * 0.0
    negi = zi + _NEG
    init = (negi, negi, negi, negi, negi, negi, zi, v)
    out = jax.lax.fori_loop(0, _NSTEPS, body, init, unroll=4)
    run_m, run_s = out[5], out[6]

    mb = jnp.max(run_m, axis=1, keepdims=True)
    sb = jnp.sum(run_s * jnp.exp(run_m - mb), axis=1, keepdims=True)
    fin = mb + jnp.log(sb)
    out_ref[...] = jnp.full((1, 1), -jnp.sum(fin) * (1.0 / _B), jnp.float32)


def _prep(predictions, targets):
    predT = jnp.transpose(predictions.astype(jnp.float32), (2, 0, 1))
    t = targets.astype(jnp.int32)
    v0 = jnp.concatenate([t[:, :1], jnp.flip(t[:, 1:], axis=1)], axis=1)
    return predT, v0


@jax.jit
def kernel(predictions, targets):
    predT, v0 = _prep(predictions, targets)
    out = pl.pallas_call(
        _sw_kernel,
        out_shape=jax.ShapeDtypeStruct((1, 1), jnp.float32),
    )(predT, v0)
    return out[0, 0]


# speedup vs baseline: 117.6585x; 1.3620x over previous
import jax
import jax.numpy as jnp
from jax.experimental import pallas as pl

_EGO = 0.01831563888873418
_EGE = 0.36787944117144233
_NEG = -1e30
_B = 16
_L = 256
_CHUNK = 32
_NCHUNK = 16


def _shiftn(x):
    return jnp.concatenate(
        [jnp.full((x.shape[0], 1), _NEG, x.dtype), x[:, :-1]], axis=1)


def _shift0(x):
    return jnp.concatenate(
        [jnp.zeros((x.shape[0], 1), x.dtype), x[:, :-1]], axis=1)


def _rot1(x):
    return jnp.concatenate([x[:, -1:], x[:, :-1]], axis=1)


def _sel4(t, v):
    return jnp.where(t == 0, v[0],
           jnp.where(t == 1, v[1],
           jnp.where(t == 2, v[2], v[3])))


def _sw_kernel(predT_ref, v0_ref, out_ref):
    predT = predT_ref[...]
    v = v0_ref[...]
    lane = jax.lax.broadcasted_iota(jnp.int32, (_B, _L), 1)
    vmask = lane < (_L - 1)
    zero = jnp.zeros((_B, _L), jnp.float32)

    p0p = [jnp.where(vmask, jnp.maximum(predT[p], 0.0), zero) for p in range(4)]
    ep0 = [jnp.where(vmask, jnp.exp(predT[p]), zero) for p in range(4)]
    ep1 = [jnp.where(vmask,
                     jnp.exp(jnp.concatenate(
                         [predT[p][:, 1:], predT[p][:, :1]], axis=1)),
                     zero) for p in range(4)]

    def step(d_base, k, carry):
        vc, mx1, mx2, ea1, er1, sq1, eg1, eg2, acc = carry
        vb = _rot1(vc)
        ld = lane - d_base - k
        mask = (ld <= 0) & (ld >= -254)
        esmx = jnp.where(mask, _sel4(vc, ep0), zero)
        sp = jnp.where(mask, _sel4(vc, p0p), zero)
        exe = jnp.where(mask, _sel4(vb, ep1), zero)
        shm1 = _shiftn(mx1)
        shm2 = _shiftn(mx2)
        mx0 = jnp.maximum(jnp.maximum(shm2 + sp, shm1), mx1)
        d1 = jnp.exp(mx1 - mx0)
        dg = jnp.exp(shm2 - mx0)
        dq = jnp.exp(shm1 - mx0)
        e0 = jnp.exp(-mx0)
        ea0 = esmx * (_shift0(eg2) * dg + e0)
        er0 = (_EGO * ea1 + _EGE * er1) * d1
        ed0 = _shift0(sq1) * dq
        eg0 = ea0 + er0 + ed0
        sq0 = _EGO * (ea0 + er0) + _EGE * ed0
        acc0 = acc * d1 + eg0 * exe
        return (vb, mx0, mx1, ea0, er0, sq0, eg0, eg1, acc0)

    def chunk(i, carry):
        d_base = i * _CHUNK
        for k in range(_CHUNK):
            carry = step(d_base, k, carry)
        vc, mx1, mx2, ea1, er1, sq1, eg1, eg2, acc = carry
        n1 = jnp.maximum(eg1, 1.0)
        r1 = 1.0 / n1
        mx1 = mx1 + jnp.log(n1)
        n2 = jnp.maximum(eg2, 1.0)
        mx2 = mx2 + jnp.log(n2)
        return (vc, mx1, mx2, ea1 * r1, er1 * r1, sq1 * r1, eg1 * r1,
                eg2 / n2, acc * r1)

    zi = predT[0] * 0.0
    init = (v, zi, zi, zi, zi, zi, zi, zi, zi)
    out = jax.lax.fori_loop(0, _NCHUNK, chunk, init)
    mx1, acc = out[1], out[8]

    t = mx1 + jnp.log(jnp.maximum(acc, 1e-35))
    mb = jnp.max(t, axis=1, keepdims=True)
    sb = jnp.sum(jnp.exp(t - mb), axis=1, keepdims=True)
    fin = mb + jnp.log(sb)
    out_ref[...] = jnp.full((1, 1), -jnp.sum(fin) * (1.0 / _B), jnp.float32)


def _prep(predictions, targets):
    predT = jnp.transpose(predictions.astype(jnp.float32), (2, 0, 1))
    t = targets.astype(jnp.int32)
    v0 = jnp.concatenate([t[:, :1], jnp.flip(t[:, 1:], axis=1)], axis=1)
    return predT, v0


@jax.jit
def kernel(predictions, targets):
    predT, v0 = _prep(predictions, targets)
    out = pl.pallas_call(
        _sw_kernel,
        out_shape=jax.ShapeDtypeStruct((1, 1), jnp.float32),
    )(predT, v0)
    return out[0, 0]
